# baseline (speedup 1.0000x reference)
import functools

import jax
import jax.numpy as jnp
from jax import lax
from jax.experimental import pallas as pl
from jax.experimental.pallas import tpu as pltpu
from jax.experimental.pallas import tpu_sc as plsc

EPS = 1e-6
RBLK = 256
NBLK = 24
BE_PAD = 16


def _cumsum_rows(a):
    n = a.shape[0]
    s = 1
    while s < n:
        shifted = jnp.concatenate([jnp.zeros((s, a.shape[1]), a.dtype), a[:-s, :]], axis=0)
        a = a + shifted
        s *= 2
    return a


def _route_body(x_ref, rmsw_ref, rw_ref, xn_ref, d0_ref, d1_ref, p0_ref, p1_ref,
                be_ref, *, n_e):
    xb = x_ref[...]
    xn = xb * jax.lax.rsqrt(jnp.mean(xb * xb, axis=-1, keepdims=True) + EPS)
    xn = xn * rmsw_ref[...]
    xn_ref[...] = xn
    scores = jnp.dot(xn, rw_ref[...].T, preferred_element_type=jnp.float32)
    s = scores.shape[0]
    iota_e = jax.lax.broadcasted_iota(jnp.int32, (s, n_e), 1)
    m0 = jnp.max(scores, axis=1, keepdims=True)
    i0 = jnp.min(jnp.where(scores == m0, iota_e, n_e), axis=1, keepdims=True)
    masked = jnp.where(iota_e == i0, -1e30, scores)
    m1 = jnp.max(masked, axis=1, keepdims=True)
    i1 = jnp.min(jnp.where(masked == m1, iota_e, n_e), axis=1, keepdims=True)
    p0 = 1.0 / (1.0 + jnp.exp(m1 - m0))
    p0_ref[...] = p0
    p1_ref[...] = 1.0 - p0

    onehot0 = (iota_e == i0).astype(jnp.int32)
    onehot1 = (iota_e == i1).astype(jnp.int32)
    c0 = _cumsum_rows(onehot0)
    c1 = _cumsum_rows(onehot1)
    counts0 = c0[s - 1:s, :]
    counts = counts0 + c1[s - 1:s, :]
    pc = ((counts + (RBLK - 1)) // RBLK) * RBLK
    eidx_r = jax.lax.broadcasted_iota(jnp.int32, (n_e, n_e), 0)
    eidx_c = jax.lax.broadcasted_iota(jnp.int32, (n_e, n_e), 1)
    tri = (eidx_r < eidx_c).astype(jnp.float32)
    poff_f = jnp.dot(pc.astype(jnp.float32), tri, preferred_element_type=jnp.float32)
    poff = poff_f.astype(jnp.int32)

    d0_ref[...] = jnp.sum(onehot0 * (poff + c0 - 1), axis=1, keepdims=True)
    d1_ref[...] = jnp.sum(onehot1 * (poff + counts0 + c1 - 1), axis=1, keepdims=True)

    poffb = poff // RBLK
    lane7 = jax.lax.broadcasted_iota(jnp.int32, (1, n_e), 1) == (n_e - 1)
    tot = jnp.sum(jnp.where(lane7, poff + pc, 0), axis=1, keepdims=True) // RBLK
    be_ref[...] = jnp.concatenate(
        [poffb, jnp.broadcast_to(tot, (1, n_e))], axis=1)


def _group_mm_body(m_ref, xg_ref, w1_ref, w2_ref, yg_ref):
    e = pl.program_id(0)
    j = pl.program_id(1)

    @pl.when(m_ref[e] + j < m_ref[e + 1])
    def _():
        xb = xg_ref[...].astype(jnp.bfloat16)
        h = jnp.dot(xb, w1_ref[0].astype(jnp.bfloat16).T,
                    preferred_element_type=jnp.float32)
        h = h * (1.0 / (1.0 + jnp.exp(-h)))
        yg_ref[...] = jnp.dot(h.astype(jnp.bfloat16), w2_ref[0].astype(jnp.bfloat16).T,
                              preferred_element_type=jnp.float32)


def _combine_body(x_ref, a_ref, b_ref, pa_ref, pb_ref, o_ref):
    o_ref[...] = x_ref[...] + pa_ref[...] * a_ref[...] + pb_ref[...] * b_ref[...]


def kernel(x, rms_w, router_w, w1, w2):
    s, d = x.shape
    n_e, d_ff, _ = w1.shape
    pt = NBLK * RBLK

    xn, d0, d1, p0, p1, be = pl.pallas_call(
        functools.partial(_route_body, n_e=n_e),
        in_specs=[
            pl.BlockSpec((s, d), lambda: (0, 0)),
            pl.BlockSpec((1, d), lambda: (0, 0)),
            pl.BlockSpec((n_e, d), lambda: (0, 0)),
        ],
        out_specs=[
            pl.BlockSpec((s, d), lambda: (0, 0)),
            pl.BlockSpec((s, 1), lambda: (0, 0)),
            pl.BlockSpec((s, 1), lambda: (0, 0)),
            pl.BlockSpec((s, 1), lambda: (0, 0)),
            pl.BlockSpec((s, 1), lambda: (0, 0)),
            pl.BlockSpec((1, BE_PAD), lambda: (0, 0)),
        ],
        out_shape=[
            jax.ShapeDtypeStruct((s, d), jnp.float32),
            jax.ShapeDtypeStruct((s, 1), jnp.int32),
            jax.ShapeDtypeStruct((s, 1), jnp.int32),
            jax.ShapeDtypeStruct((s, 1), jnp.float32),
            jax.ShapeDtypeStruct((s, 1), jnp.float32),
            jax.ShapeDtypeStruct((1, BE_PAD), jnp.int32),
        ],
    )(x, rms_w.reshape(1, d), router_w)

    d0f = d0.reshape(s)
    d1f = d1.reshape(s)

    info = plsc.get_sparse_core_info()
    nw = info.num_cores * info.num_subcores
    tpw = s // nw
    mesh = plsc.VectorSubcoreMesh(core_axis_name="c", subcore_axis_name="s")

    @functools.partial(
        pl.kernel, mesh=mesh,
        out_type=jax.ShapeDtypeStruct((pt, d), jnp.float32),
        scratch_types=[
            pltpu.VMEM((tpw,), jnp.int32),
            pltpu.VMEM((tpw, d), jnp.float32),
            pltpu.SemaphoreType.DMA,
        ],
    )
    def _scatter_k(xn_hbm, d0_hbm, d1_hbm, xg_hbm, idx_v, rows_v, sem):
        wid = lax.axis_index("s") * info.num_cores + lax.axis_index("c")
        base = wid * tpw
        pltpu.sync_copy(xn_hbm.at[pl.ds(base, tpw)], rows_v)
        pltpu.sync_copy(d0_hbm.at[pl.ds(base, tpw)], idx_v)
        pltpu.async_copy(rows_v, xg_hbm.at[idx_v], sem).wait()
        pltpu.sync_copy(d1_hbm.at[pl.ds(base, tpw)], idx_v)
        pltpu.async_copy(rows_v, xg_hbm.at[idx_v], sem).wait()

    xg = _scatter_k(xn, d0f, d1f)

    jmax = s // RBLK
    yg = pl.pallas_call(
        _group_mm_body,
        grid_spec=pltpu.PrefetchScalarGridSpec(
            num_scalar_prefetch=1,
            grid=(n_e, jmax),
            in_specs=[
                pl.BlockSpec((RBLK, d),
                             lambda e, j, m: (jnp.minimum(m[e] + j, NBLK - 1), 0)),
                pl.BlockSpec((1, d_ff, d), lambda e, j, m: (e, 0, 0)),
                pl.BlockSpec((1, d, d_ff), lambda e, j, m: (e, 0, 0)),
            ],
            out_specs=pl.BlockSpec(
                (RBLK, d),
                lambda e, j, m: (jnp.where(m[e] + j < m[e + 1], m[e] + j, NBLK), 0)),
        ),
        out_shape=jax.ShapeDtypeStruct((pt + RBLK, d), jnp.float32),
        compiler_params=pltpu.CompilerParams(
            dimension_semantics=("arbitrary", "arbitrary"),
        ),
    )(be.reshape(BE_PAD), xg, w1, w2)

    @functools.partial(
        pl.kernel, mesh=mesh,
        out_type=jax.ShapeDtypeStruct((2 * s, d), jnp.float32),
        scratch_types=[
            pltpu.VMEM((tpw,), jnp.int32),
            pltpu.VMEM((tpw, d), jnp.float32),
            pltpu.SemaphoreType.DMA,
        ],
    )
    def _gather_k(yg_hbm, d0_hbm, d1_hbm, yp_hbm, idx_v, rows_v, sem):
        wid = lax.axis_index("s") * info.num_cores + lax.axis_index("c")
        base = wid * tpw
        pltpu.sync_copy(d0_hbm.at[pl.ds(base, tpw)], idx_v)
        pltpu.async_copy(yg_hbm.at[idx_v], rows_v, sem).wait()
        pltpu.sync_copy(rows_v, yp_hbm.at[pl.ds(base, tpw)])
        pltpu.sync_copy(d1_hbm.at[pl.ds(base, tpw)], idx_v)
        pltpu.async_copy(yg_hbm.at[idx_v], rows_v, sem).wait()
        pltpu.sync_copy(rows_v, yp_hbm.at[pl.ds(s + base, tpw)])

    yp = _gather_k(yg, d0f, d1f)

    eblk = 256
    out = pl.pallas_call(
        _combine_body,
        grid=(s // eblk,),
        in_specs=[
            pl.BlockSpec((eblk, d), lambda r: (r, 0)),
            pl.BlockSpec((eblk, d), lambda r: (r, 0)),
            pl.BlockSpec((eblk, d), lambda r: (r + s // eblk, 0)),
            pl.BlockSpec((eblk, 1), lambda r: (r, 0)),
            pl.BlockSpec((eblk, 1), lambda r: (r, 0)),
        ],
        out_specs=pl.BlockSpec((eblk, d), lambda r: (r, 0)),
        out_shape=jax.ShapeDtypeStruct((s, d), jnp.float32),
    )(x, yp, yp, p0, p1)
    return out


# speedup vs baseline: 1.0891x; 1.0891x over previous
import functools

import jax
import jax.numpy as jnp
from jax import lax
from jax.experimental import pallas as pl
from jax.experimental.pallas import tpu as pltpu
from jax.experimental.pallas import tpu_sc as plsc

EPS = 1e-6
RBLK = 256
NBLK = 24
BE_PAD = 16


def _cumsum_rows(a):
    n = a.shape[0]
    s = 1
    while s < n:
        shifted = jnp.concatenate([jnp.zeros((s, a.shape[1]), a.dtype), a[:-s, :]], axis=0)
        a = a + shifted
        s *= 2
    return a


def _route_body(x_ref, rmsw_ref, rw_ref, xn_ref, d0_ref, d1_ref, p0_ref, p1_ref,
                be_ref, *, n_e):
    xb = x_ref[...]
    xn = xb * jax.lax.rsqrt(jnp.mean(xb * xb, axis=-1, keepdims=True) + EPS)
    xn = xn * rmsw_ref[...]
    xn_ref[...] = xn
    scores = jnp.dot(xn, rw_ref[...].T, preferred_element_type=jnp.float32)
    s = scores.shape[0]
    iota_e = jax.lax.broadcasted_iota(jnp.int32, (s, n_e), 1)
    m0 = jnp.max(scores, axis=1, keepdims=True)
    i0 = jnp.min(jnp.where(scores == m0, iota_e, n_e), axis=1, keepdims=True)
    masked = jnp.where(iota_e == i0, -1e30, scores)
    m1 = jnp.max(masked, axis=1, keepdims=True)
    i1 = jnp.min(jnp.where(masked == m1, iota_e, n_e), axis=1, keepdims=True)
    p0 = 1.0 / (1.0 + jnp.exp(m1 - m0))
    p0_ref[...] = p0
    p1_ref[...] = 1.0 - p0

    onehot0 = (iota_e == i0).astype(jnp.int32)
    onehot1 = (iota_e == i1).astype(jnp.int32)
    c0 = _cumsum_rows(onehot0)
    c1 = _cumsum_rows(onehot1)
    counts0 = c0[s - 1:s, :]
    counts = counts0 + c1[s - 1:s, :]
    pc = ((counts + (RBLK - 1)) // RBLK) * RBLK
    eidx_r = jax.lax.broadcasted_iota(jnp.int32, (n_e, n_e), 0)
    eidx_c = jax.lax.broadcasted_iota(jnp.int32, (n_e, n_e), 1)
    tri = (eidx_r < eidx_c).astype(jnp.float32)
    poff_f = jnp.dot(pc.astype(jnp.float32), tri, preferred_element_type=jnp.float32)
    poff = poff_f.astype(jnp.int32)

    d0_ref[...] = jnp.sum(onehot0 * (poff + c0 - 1), axis=1, keepdims=True)
    d1_ref[...] = jnp.sum(onehot1 * (poff + counts0 + c1 - 1), axis=1, keepdims=True)

    poffb = poff // RBLK
    lane7 = jax.lax.broadcasted_iota(jnp.int32, (1, n_e), 1) == (n_e - 1)
    tot = jnp.sum(jnp.where(lane7, poff + pc, 0), axis=1, keepdims=True) // RBLK
    be_ref[...] = jnp.concatenate(
        [poffb, jnp.broadcast_to(tot, (1, n_e))], axis=1)


FCH = 768
JMAX = 8


def _group_mm_body(m_ref, xg_hbm, w1_hbm, w2_hbm, yg_hbm,
                   xbuf, yacc, w1b, w2b, w1c16, w2c16,
                   xsem, wsem, ysem, *, n_ch):
    e = pl.program_id(0)
    start = m_ref[e]
    nb = m_ref[e + 1] - m_ref[e]

    def x_copy(j):
        return pltpu.make_async_copy(
            xg_hbm.at[pl.ds((start + j) * RBLK, RBLK), :],
            xbuf.at[pl.ds(j * RBLK, RBLK), :], xsem.at[j])

    def w_copies(c, slot):
        return (
            pltpu.make_async_copy(w1_hbm.at[e, pl.ds(c * FCH, FCH), :],
                                  w1b.at[slot], wsem.at[slot, 0]),
            pltpu.make_async_copy(w2_hbm.at[e, :, pl.ds(c * FCH, FCH)],
                                  w2b.at[slot], wsem.at[slot, 1]),
        )

    def y_copy(j):
        return pltpu.make_async_copy(
            yacc.at[pl.ds(j * RBLK, RBLK), :],
            yg_hbm.at[pl.ds((start + j) * RBLK, RBLK), :], ysem.at[j])

    for j in range(JMAX):
        @pl.when(j < nb)
        def _(j=j):
            x_copy(j).start()

    @pl.when(nb > 0)
    def _():
        a, b = w_copies(0, 0)
        a.start()
        b.start()

    for j in range(JMAX):
        @pl.when(j < nb)
        def _(j=j):
            x_copy(j).wait()

    for c in range(n_ch):
        slot = c % 2

        @pl.when(nb > 0)
        def _(c=c, slot=slot):
            if c + 1 < n_ch:
                a, b = w_copies(c + 1, 1 - slot)
                a.start()
                b.start()
            a0, b0 = w_copies(c, slot)
            a0.wait()
            b0.wait()
            w1c16[...] = w1b[slot].astype(jnp.bfloat16)
            w2c16[...] = w2b[slot].astype(jnp.bfloat16)

        for j in range(JMAX):
            @pl.when(j < nb)
            def _(c=c, j=j):
                xb = xbuf[pl.ds(j * RBLK, RBLK), :].astype(jnp.bfloat16)
                h = jnp.dot(xb, w1c16[...].T, preferred_element_type=jnp.float32)
                h = h * (1.0 / (1.0 + jnp.exp(-h)))
                yc = jnp.dot(h.astype(jnp.bfloat16), w2c16[...].T,
                             preferred_element_type=jnp.float32)
                if c == 0:
                    yacc[pl.ds(j * RBLK, RBLK), :] = yc
                else:
                    yacc[pl.ds(j * RBLK, RBLK), :] += yc

    for j in range(JMAX):
        @pl.when(j < nb)
        def _(j=j):
            y_copy(j).start()
    for j in range(JMAX):
        @pl.when(j < nb)
        def _(j=j):
            y_copy(j).wait()


def _combine_body(x_ref, a_ref, b_ref, pa_ref, pb_ref, o_ref):
    o_ref[...] = x_ref[...] + pa_ref[...] * a_ref[...] + pb_ref[...] * b_ref[...]


def kernel(x, rms_w, router_w, w1, w2):
    s, d = x.shape
    n_e, d_ff, _ = w1.shape
    pt = NBLK * RBLK

    xn, d0, d1, p0, p1, be = pl.pallas_call(
        functools.partial(_route_body, n_e=n_e),
        in_specs=[
            pl.BlockSpec((s, d), lambda: (0, 0)),
            pl.BlockSpec((1, d), lambda: (0, 0)),
            pl.BlockSpec((n_e, d), lambda: (0, 0)),
        ],
        out_specs=[
            pl.BlockSpec((s, d), lambda: (0, 0)),
            pl.BlockSpec((s, 1), lambda: (0, 0)),
            pl.BlockSpec((s, 1), lambda: (0, 0)),
            pl.BlockSpec((s, 1), lambda: (0, 0)),
            pl.BlockSpec((s, 1), lambda: (0, 0)),
            pl.BlockSpec((1, BE_PAD), lambda: (0, 0)),
        ],
        out_shape=[
            jax.ShapeDtypeStruct((s, d), jnp.float32),
            jax.ShapeDtypeStruct((s, 1), jnp.int32),
            jax.ShapeDtypeStruct((s, 1), jnp.int32),
            jax.ShapeDtypeStruct((s, 1), jnp.float32),
            jax.ShapeDtypeStruct((s, 1), jnp.float32),
            jax.ShapeDtypeStruct((1, BE_PAD), jnp.int32),
        ],
    )(x, rms_w.reshape(1, d), router_w)

    d0f = d0.reshape(s)
    d1f = d1.reshape(s)

    info = plsc.get_sparse_core_info()
    nw = info.num_cores * info.num_subcores
    tpw = s // nw
    mesh = plsc.VectorSubcoreMesh(core_axis_name="c", subcore_axis_name="s")

    @functools.partial(
        pl.kernel, mesh=mesh,
        out_type=jax.ShapeDtypeStruct((pt, d), jnp.float32),
        scratch_types=[
            pltpu.VMEM((tpw,), jnp.int32),
            pltpu.VMEM((tpw, d), jnp.float32),
            pltpu.SemaphoreType.DMA,
        ],
    )
    def _scatter_k(xn_hbm, d0_hbm, d1_hbm, xg_hbm, idx_v, rows_v, sem):
        wid = lax.axis_index("s") * info.num_cores + lax.axis_index("c")
        base = wid * tpw
        pltpu.sync_copy(xn_hbm.at[pl.ds(base, tpw)], rows_v)
        pltpu.sync_copy(d0_hbm.at[pl.ds(base, tpw)], idx_v)
        pltpu.async_copy(rows_v, xg_hbm.at[idx_v], sem).wait()
        pltpu.sync_copy(d1_hbm.at[pl.ds(base, tpw)], idx_v)
        pltpu.async_copy(rows_v, xg_hbm.at[idx_v], sem).wait()

    xg = _scatter_k(xn, d0f, d1f)

    n_ch = d_ff // FCH
    yg = pl.pallas_call(
        functools.partial(_group_mm_body, n_ch=n_ch),
        grid_spec=pltpu.PrefetchScalarGridSpec(
            num_scalar_prefetch=1,
            grid=(n_e,),
            in_specs=[
                pl.BlockSpec(memory_space=pltpu.MemorySpace.HBM),
                pl.BlockSpec(memory_space=pltpu.MemorySpace.HBM),
                pl.BlockSpec(memory_space=pltpu.MemorySpace.HBM),
            ],
            out_specs=pl.BlockSpec(memory_space=pltpu.MemorySpace.HBM),
            scratch_shapes=[
                pltpu.VMEM((JMAX * RBLK, d), jnp.float32),
                pltpu.VMEM((JMAX * RBLK, d), jnp.float32),
                pltpu.VMEM((2, FCH, d), jnp.float32),
                pltpu.VMEM((2, d, FCH), jnp.float32),
                pltpu.VMEM((FCH, d), jnp.bfloat16),
                pltpu.VMEM((d, FCH), jnp.bfloat16),
                pltpu.SemaphoreType.DMA((JMAX,)),
                pltpu.SemaphoreType.DMA((2, 2)),
                pltpu.SemaphoreType.DMA((JMAX,)),
            ],
        ),
        out_shape=jax.ShapeDtypeStruct((pt, d), jnp.float32),
        compiler_params=pltpu.CompilerParams(
            dimension_semantics=("arbitrary",),
        ),
    )(be.reshape(BE_PAD), xg, w1, w2)

    @functools.partial(
        pl.kernel, mesh=mesh,
        out_type=jax.ShapeDtypeStruct((2 * s, d), jnp.float32),
        scratch_types=[
            pltpu.VMEM((tpw,), jnp.int32),
            pltpu.VMEM((tpw, d), jnp.float32),
            pltpu.SemaphoreType.DMA,
        ],
    )
    def _gather_k(yg_hbm, d0_hbm, d1_hbm, yp_hbm, idx_v, rows_v, sem):
        wid = lax.axis_index("s") * info.num_cores + lax.axis_index("c")
        base = wid * tpw
        pltpu.sync_copy(d0_hbm.at[pl.ds(base, tpw)], idx_v)
        pltpu.async_copy(yg_hbm.at[idx_v], rows_v, sem).wait()
        pltpu.sync_copy(rows_v, yp_hbm.at[pl.ds(base, tpw)])
        pltpu.sync_copy(d1_hbm.at[pl.ds(base, tpw)], idx_v)
        pltpu.async_copy(yg_hbm.at[idx_v], rows_v, sem).wait()
        pltpu.sync_copy(rows_v, yp_hbm.at[pl.ds(s + base, tpw)])

    yp = _gather_k(yg, d0f, d1f)

    eblk = 256
    out = pl.pallas_call(
        _combine_body,
        grid=(s // eblk,),
        in_specs=[
            pl.BlockSpec((eblk, d), lambda r: (r, 0)),
            pl.BlockSpec((eblk, d), lambda r: (r, 0)),
            pl.BlockSpec((eblk, d), lambda r: (r + s // eblk, 0)),
            pl.BlockSpec((eblk, 1), lambda r: (r, 0)),
            pl.BlockSpec((eblk, 1), lambda r: (r, 0)),
        ],
        out_specs=pl.BlockSpec((eblk, d), lambda r: (r, 0)),
        out_shape=jax.ShapeDtypeStruct((s, d), jnp.float32),
    )(x, yp, yp, p0, p1)
    return out
